# baseline (speedup 1.0000x reference)
import jax
import jax.numpy as jnp
import numpy as np
from jax import lax
from jax.experimental import pallas as pl
from jax.experimental.pallas import tpu as pltpu

R = 6

_GUMBEL = np.array([
    float.fromhex("0x1.561c940000000p-2"),
    float.fromhex("0x1.e76f180000000p-1"),
    float.fromhex("0x1.7378be0000000p-1"),
    float.fromhex("0x1.18a9f00000000p-1"),
    float.fromhex("0x1.d07f1e0000000p-3"),
    float.fromhex("0x1.4092440000000p-1"),
], dtype=np.float32)


def _body(x_ref, w_ref, b_ref, o_ref, best_ref, bidx_ref):
    row = pl.program_id(0)

    @pl.when(row == 0)
    def _init():
        best_ref[0] = jnp.float32(-jnp.inf)
        bidx_ref[0] = jnp.int32(0)

    g_row = jnp.float32(_GUMBEL[R - 1])
    for t in range(R - 1):
        g_row = jnp.where(row == t, jnp.float32(_GUMBEL[t]), g_row)
    s_row = jnp.sum(w_ref[0] * x_ref[...]) + b_ref[row] + g_row
    take = s_row > best_ref[0]
    bidx_ref[0] = jnp.where(take, row, bidx_ref[0])
    best_ref[0] = jnp.where(take, s_row, best_ref[0])

    @pl.when(row == R - 1)
    def _fin():
        o_ref[0] = bidx_ref[0]


_ROUTER = pl.pallas_call(
    _body,
    grid=(R,),
    out_shape=jax.ShapeDtypeStruct((1,), jnp.int32),
    in_specs=[
        pl.BlockSpec((16, 128), lambda r: (0, 0)),
        pl.BlockSpec((1, 16, 128), lambda r: (r, 0, 0)),
        pl.BlockSpec(memory_space=pltpu.SMEM),
    ],
    out_specs=pl.BlockSpec(memory_space=pltpu.SMEM),
    scratch_shapes=[
        pltpu.SMEM((1,), jnp.float32),
        pltpu.SMEM((1,), jnp.int32),
    ],
)


def kernel(X, W, b):
    x2 = jnp.reshape(X, (16, 128))
    w3 = jnp.reshape(W, (R, 16, 128))
    idx = _ROUTER(x2, w3, b)
    return idx.astype(jnp.int64)


# speedup vs baseline: 1.4252x; 1.4252x over previous
import jax
import jax.numpy as jnp
import numpy as np
from jax import lax
from jax.experimental import pallas as pl
from jax.experimental.pallas import tpu as pltpu

K = 2048
R = 6

_GUMBEL = np.array([
    float.fromhex("0x1.561c940000000p-2"),
    float.fromhex("0x1.e76f180000000p-1"),
    float.fromhex("0x1.7378be0000000p-1"),
    float.fromhex("0x1.18a9f00000000p-1"),
    float.fromhex("0x1.d07f1e0000000p-3"),
    float.fromhex("0x1.4092440000000p-1"),
], dtype=np.float32)


def _body(x_ref, w_ref, b_ref, o_ref):
    xr = x_ref[...]
    best = jnp.float32(-jnp.inf)
    bidx = jnp.int32(0)
    for row in range(R):
        s_row = jnp.sum(w_ref[row] * xr) + b_ref[row] + jnp.float32(
            _GUMBEL[row])
        take = s_row > best
        bidx = jnp.where(take, jnp.int32(row), bidx)
        best = jnp.where(take, s_row, best)
    o_ref[0] = bidx


_ROUTER = pl.pallas_call(
    _body,
    out_shape=jax.ShapeDtypeStruct((1,), jnp.int32),
    in_specs=[
        pl.BlockSpec(memory_space=pltpu.VMEM),
        pl.BlockSpec(memory_space=pltpu.VMEM),
        pl.BlockSpec(memory_space=pltpu.SMEM),
    ],
    out_specs=pl.BlockSpec(memory_space=pltpu.SMEM),
)


def kernel(X, W, b):
    x2 = jnp.reshape(X, (16, 128))
    w3 = jnp.reshape(W, (R, 16, 128))
    idx = _ROUTER(x2, w3, b)
    return idx.astype(jnp.int64)


# speedup vs baseline: 1.4358x; 1.0074x over previous
import jax
import jax.numpy as jnp
import numpy as np
from jax import lax
from jax.experimental import pallas as pl
from jax.experimental.pallas import tpu as pltpu

K = 2048
R = 6

_GUMBEL = np.array([
    float.fromhex("0x1.561c940000000p-2"),
    float.fromhex("0x1.e76f180000000p-1"),
    float.fromhex("0x1.7378be0000000p-1"),
    float.fromhex("0x1.18a9f00000000p-1"),
    float.fromhex("0x1.d07f1e0000000p-3"),
    float.fromhex("0x1.4092440000000p-1"),
], dtype=np.float32)


def _body(x_ref, w_ref, b_ref, o_ref):
    xr = x_ref[...]
    best = jnp.float32(-jnp.inf)
    bidx = jnp.int32(0)
    for row in range(R):
        s_row = (jnp.sum(w_ref[row] * xr) + b_ref[0, row]
                 + jnp.float32(_GUMBEL[row]))
        take = s_row > best
        bidx = jnp.where(take, jnp.int32(row), bidx)
        best = jnp.where(take, s_row, best)
    o_ref[0] = bidx


_ROUTER = pl.pallas_call(
    _body,
    out_shape=jax.ShapeDtypeStruct((1,), jnp.int32),
    in_specs=[
        pl.BlockSpec(memory_space=pltpu.VMEM),
        pl.BlockSpec(memory_space=pltpu.VMEM),
        pl.BlockSpec(memory_space=pltpu.VMEM),
    ],
    out_specs=pl.BlockSpec(memory_space=pltpu.SMEM),
)


def kernel(X, W, b):
    x2 = jnp.reshape(X, (16, 128))
    w3 = jnp.reshape(W, (R, 16, 128))
    idx = _ROUTER(x2, w3, jnp.reshape(b, (1, R)))
    return idx.astype(jnp.int64)
